# baseline (speedup 1.0000x reference)
import functools

import jax
import jax.numpy as jnp
import numpy as np
from jax import lax
from jax.experimental import pallas as pl
from jax.experimental.pallas import tpu as pltpu
from jax.experimental.pallas import tpu_sc as plsc

NUM_FIELDS = 26
VOCAB = 1000
EMB_DIM = 50
NUM_NUMERIC = 13
BATCH = 16384
H1 = 128

NC = 2
NS = 16
NW = NC * NS
BPW = BATCH // NW
LPW = BPW * NUM_FIELDS
RPC = 4
CW = RPC * NUM_FIELDS
NCHUNK = LPW // CW
LANES = 16
FOFF_P = 208


def _fold_body(e_ref, w_ref, o_ref):
    o_ref[...] = jnp.dot(e_ref[0], w_ref[0], preferred_element_type=jnp.float32)


def _fold_tables(emb_tables, w1_emb):
    return pl.pallas_call(
        _fold_body,
        grid=(NUM_FIELDS,),
        in_specs=[
            pl.BlockSpec((1, VOCAB, EMB_DIM), lambda f: (f, 0, 0)),
            pl.BlockSpec((1, EMB_DIM, H1), lambda f: (f, 0, 0)),
        ],
        out_specs=pl.BlockSpec((VOCAB, H1), lambda f: (f, 0)),
        out_shape=jax.ShapeDtypeStruct((NUM_FIELDS * VOCAB, H1), jnp.float32),
    )(emb_tables, w1_emb)


NBUF = 3


def _gather_sum_body(t_hbm, xc_hbm, pat_hbm, foff_hbm, zeros_hbm, out_hbm,
                     idx_v, pat_v, foff_v, bufs, acc, sgs):
    cid = lax.axis_index("c")
    sid = lax.axis_index("s")
    wid = sid * NC + cid
    pltpu.sync_copy(xc_hbm.at[wid], idx_v)
    pltpu.sync_copy(pat_hbm, pat_v)
    pltpu.sync_copy(foff_hbm, foff_v)

    def add_off(gi, _):
        s0 = gi * FOFF_P
        for j in range(FOFF_P // LANES):
            sl = pl.ds(s0 + j * LANES, LANES)
            idx_v[sl] = idx_v[sl] + foff_v[pl.ds(j * LANES, LANES)]
        return 0

    lax.fori_loop(0, LPW // FOFF_P, add_off, 0)
    my_acc = acc.at[pl.ds(sid * BPW, BPW)]
    pltpu.sync_copy(zeros_hbm, my_acc)

    for k in range(NBUF):
        pltpu.async_copy(t_hbm.at[idx_v.at[pl.ds(k * CW, CW)]], bufs[k], sgs[k])

    def body(cb, _):
        c = cb * NBUF
        for k in range(NBUF):
            pltpu.make_async_copy(t_hbm.at[idx_v.at[pl.ds((c + k) * CW, CW)]], bufs[k],
                                  sgs[k]).wait()
            row0 = sid * BPW + (c + k) * RPC
            pltpu.sync_copy(bufs[k], acc.at[pl.ds(row0, RPC)].at[pat_v],
                            add=True)
            nxt = lax.rem(c + NBUF + k, NCHUNK)
            pltpu.async_copy(t_hbm.at[idx_v.at[pl.ds(nxt * CW, CW)]], bufs[k], sgs[k])
        return 0

    lax.fori_loop(0, NCHUNK // NBUF, body, 0)
    done = (NCHUNK // NBUF) * NBUF
    for k in range(NBUF):
        ch = done + k
        pltpu.make_async_copy(t_hbm.at[idx_v.at[pl.ds(k * CW, CW)]], bufs[k], sgs[k]).wait()
        if ch < NCHUNK:
            row0 = sid * BPW + ch * RPC
            pltpu.sync_copy(bufs[k], acc.at[pl.ds(row0, RPC)].at[pat_v],
                            add=True)

    pltpu.sync_copy(my_acc, out_hbm.at[pl.ds(wid * BPW, BPW)])


def _gather_sum(t_flat, xc2, pat, foff, zeros):
    mesh = plsc.VectorSubcoreMesh(core_axis_name="c", subcore_axis_name="s",
                                  num_cores=NC, num_subcores=NS)
    f = functools.partial(
        pl.kernel,
        out_type=jax.ShapeDtypeStruct((BATCH, H1), jnp.float32),
        mesh=mesh,
        scratch_types=[
            pltpu.VMEM((LPW,), jnp.int32),
            pltpu.VMEM((CW,), jnp.int32),
            pltpu.VMEM((FOFF_P,), jnp.int32),
            [pltpu.VMEM((CW, H1), jnp.float32) for _ in range(NBUF)],
            pltpu.VMEM_SHARED((NS * BPW, H1), jnp.float32),
            [pltpu.SemaphoreType.DMA for _ in range(NBUF)],
        ],
    )(_gather_sum_body)
    return f(t_flat, xc2, pat, foff, zeros)


def _mlp_body(g_ref, xn_ref, w1n_ref, b1_ref, w2_ref, b2_ref, w3_ref, b3_ref,
              w4_ref, b4_ref, o_ref):
    h = g_ref[...] + jnp.dot(xn_ref[...], w1n_ref[...],
                             preferred_element_type=jnp.float32)
    h = jax.nn.relu(h + b1_ref[...])
    h = jax.nn.relu(jnp.dot(h, w2_ref[...], preferred_element_type=jnp.float32)
                    + b2_ref[...])
    h = jax.nn.relu(jnp.dot(h, w3_ref[...], preferred_element_type=jnp.float32)
                    + b3_ref[...])
    o_ref[...] = (jnp.dot(h, w4_ref[...], preferred_element_type=jnp.float32)
                  + b4_ref[...])


def _mlp_tail(g, x_num, w1n, b1, w2, b2, w3, b3, w4, b4):
    BB = 2048
    full = lambda *shape: pl.BlockSpec(shape, lambda i: (0,) * len(shape))
    return pl.pallas_call(
        _mlp_body,
        grid=(BATCH // BB,),
        in_specs=[
            pl.BlockSpec((BB, H1), lambda i: (i, 0)),
            pl.BlockSpec((BB, NUM_NUMERIC), lambda i: (i, 0)),
            full(NUM_NUMERIC, H1),
            full(H1),
            full(H1, 64),
            full(64),
            full(64, 32),
            full(32),
            full(32, 1),
            full(1),
        ],
        out_specs=pl.BlockSpec((BB, 1), lambda i: (i, 0)),
        out_shape=jax.ShapeDtypeStruct((BATCH, 1), jnp.float32),
    )(g, x_num, w1n, b1, w2, b2, w3, b3, w4, b4)


_PAT = np.arange(CW, dtype=np.int32) // NUM_FIELDS
_ZEROS = np.zeros((BPW, H1), np.float32)
_FOFF = (np.arange(208, dtype=np.int32) % NUM_FIELDS) * VOCAB


def kernel(x_num, x_cat, emb_tables, W1, b1, W2, b2, W3, b3, W4, b4):
    w1_num = W1[:NUM_NUMERIC]
    w1_emb = W1[NUM_NUMERIC:].reshape(NUM_FIELDS, EMB_DIM, H1)

    t_flat = _fold_tables(emb_tables, w1_emb)

    xc2 = x_cat.reshape(NW, LPW)

    g = _gather_sum(t_flat, xc2, _PAT, _FOFF, _ZEROS)

    out = _mlp_tail(g, x_num, w1_num, b1, W2, b2, W3, b3, W4, b4)
    return out[:, 0]


# speedup vs baseline: 1.0208x; 1.0208x over previous
import functools

import jax
import jax.numpy as jnp
import numpy as np
from jax import lax
from jax.experimental import pallas as pl
from jax.experimental.pallas import tpu as pltpu
from jax.experimental.pallas import tpu_sc as plsc

NUM_FIELDS = 26
VOCAB = 1000
EMB_DIM = 50
NUM_NUMERIC = 13
BATCH = 16384
H1 = 128

NC = 2
NS = 16
NW = NC * NS
BPW = BATCH // NW
LPW = BPW * NUM_FIELDS
RPC = 4
CW = RPC * NUM_FIELDS
NCHUNK = LPW // CW
LANES = 16


def _fold_body(e_ref, w_ref, o_ref):
    o_ref[...] = jnp.dot(e_ref[0], w_ref[0], preferred_element_type=jnp.float32)


def _fold_tables(emb_tables, w1_emb):
    return pl.pallas_call(
        _fold_body,
        grid=(NUM_FIELDS,),
        in_specs=[
            pl.BlockSpec((1, VOCAB, EMB_DIM), lambda f: (f, 0, 0)),
            pl.BlockSpec((1, EMB_DIM, H1), lambda f: (f, 0, 0)),
        ],
        out_specs=pl.BlockSpec((VOCAB, H1), lambda f: (f, 0)),
        out_shape=jax.ShapeDtypeStruct((NUM_FIELDS * VOCAB, H1), jnp.float32),
    )(emb_tables, w1_emb)


NBUF = 3


def _gather_sum_body(t_hbm, idx_hbm, pat_hbm, zeros_hbm, out_hbm,
                     idx_v, pat_v, bufs, acc, sgs):
    cid = lax.axis_index("c")
    sid = lax.axis_index("s")
    wid = sid * NC + cid
    pltpu.sync_copy(idx_hbm.at[wid], idx_v)
    pltpu.sync_copy(pat_hbm, pat_v)
    my_acc = acc.at[pl.ds(sid * BPW, BPW)]
    pltpu.sync_copy(zeros_hbm, my_acc)

    for k in range(NBUF):
        pltpu.async_copy(t_hbm.at[idx_v.at[k]], bufs[k], sgs[k])

    def body(cb, _):
        c = cb * NBUF
        for k in range(NBUF):
            pltpu.make_async_copy(t_hbm.at[idx_v.at[c + k]], bufs[k],
                                  sgs[k]).wait()
            row0 = sid * BPW + (c + k) * RPC
            pltpu.sync_copy(bufs[k], acc.at[pl.ds(row0, RPC)].at[pat_v],
                            add=True)
            nxt = lax.rem(c + NBUF + k, NCHUNK)
            pltpu.async_copy(t_hbm.at[idx_v.at[nxt]], bufs[k], sgs[k])
        return 0

    lax.fori_loop(0, NCHUNK // NBUF, body, 0)
    done = (NCHUNK // NBUF) * NBUF
    for k in range(NBUF):
        ch = done + k
        pltpu.make_async_copy(t_hbm.at[idx_v.at[k]], bufs[k], sgs[k]).wait()
        if ch < NCHUNK:
            row0 = sid * BPW + ch * RPC
            pltpu.sync_copy(bufs[k], acc.at[pl.ds(row0, RPC)].at[pat_v],
                            add=True)

    pltpu.sync_copy(my_acc, out_hbm.at[pl.ds(wid * BPW, BPW)])


def _gather_sum(t_flat, idx3, pat, zeros):
    mesh = plsc.VectorSubcoreMesh(core_axis_name="c", subcore_axis_name="s",
                                  num_cores=NC, num_subcores=NS)
    f = functools.partial(
        pl.kernel,
        out_type=jax.ShapeDtypeStruct((BATCH, H1), jnp.float32),
        mesh=mesh,
        scratch_types=[
            pltpu.VMEM((NCHUNK, CW), jnp.int32),
            pltpu.VMEM((CW,), jnp.int32),
            [pltpu.VMEM((CW, H1), jnp.float32) for _ in range(NBUF)],
            pltpu.VMEM_SHARED((NS * BPW, H1), jnp.float32),
            [pltpu.SemaphoreType.DMA for _ in range(NBUF)],
        ],
    )(_gather_sum_body)
    return f(t_flat, idx3, pat, zeros)


def _mlp_body(g_ref, xn_ref, w1n_ref, b1_ref, w2_ref, b2_ref, w3_ref, b3_ref,
              w4_ref, b4_ref, o_ref):
    h = g_ref[...] + jnp.dot(xn_ref[...], w1n_ref[...],
                             preferred_element_type=jnp.float32)
    h = jax.nn.relu(h + b1_ref[...])
    h = jax.nn.relu(jnp.dot(h, w2_ref[...], preferred_element_type=jnp.float32)
                    + b2_ref[...])
    h = jax.nn.relu(jnp.dot(h, w3_ref[...], preferred_element_type=jnp.float32)
                    + b3_ref[...])
    o_ref[...] = (jnp.dot(h, w4_ref[...], preferred_element_type=jnp.float32)
                  + b4_ref[...])


def _mlp_tail(g, x_num, w1n, b1, w2, b2, w3, b3, w4, b4):
    BB = 4096
    full = lambda *shape: pl.BlockSpec(shape, lambda i: (0,) * len(shape))
    return pl.pallas_call(
        _mlp_body,
        grid=(BATCH // BB,),
        in_specs=[
            pl.BlockSpec((BB, H1), lambda i: (i, 0)),
            pl.BlockSpec((BB, NUM_NUMERIC), lambda i: (i, 0)),
            full(NUM_NUMERIC, H1),
            full(H1),
            full(H1, 64),
            full(64),
            full(64, 32),
            full(32),
            full(32, 1),
            full(1),
        ],
        out_specs=pl.BlockSpec((BB, 1), lambda i: (i, 0)),
        out_shape=jax.ShapeDtypeStruct((BATCH, 1), jnp.float32),
    )(g, x_num, w1n, b1, w2, b2, w3, b3, w4, b4)


_PAT = np.arange(CW, dtype=np.int32) // NUM_FIELDS
_ZEROS = np.zeros((BPW, H1), np.float32)
_FIELD_OFF = np.arange(NUM_FIELDS, dtype=np.int32) * VOCAB


def kernel(x_num, x_cat, emb_tables, W1, b1, W2, b2, W3, b3, W4, b4):
    w1_num = W1[:NUM_NUMERIC]
    w1_emb = W1[NUM_NUMERIC:].reshape(NUM_FIELDS, EMB_DIM, H1)

    t_flat = _fold_tables(emb_tables, w1_emb)

    idx = x_cat + _FIELD_OFF[None, :]
    idx3 = idx.reshape(NW, NCHUNK, CW)

    g = _gather_sum(t_flat, idx3, _PAT, _ZEROS)

    out = _mlp_tail(g, x_num, w1_num, b1, W2, b2, W3, b3, W4, b4)
    return out[:, 0]
